# baseline (speedup 1.0000x reference)
import functools

import jax
import jax.numpy as jnp
from jax.experimental import pallas as pl
from jax.experimental.pallas import tpu as pltpu

_TAU0 = 0.5
_K = 16


def _body(logits_ref, uniform_ref, out_ref):
    l = logits_ref[...]
    m = jnp.max(l, axis=-1, keepdims=True)
    e = jnp.exp(2.0 * (l - m))
    g = jnp.log2(jnp.maximum(uniform_ref[...], 1e-10))
    n = e[:, None, :] * pl.reciprocal(g * g, approx=True, full_range=False)
    s = jnp.sum(n, axis=-1, keepdims=True)
    r = pl.reciprocal(s, approx=True, full_range=False)
    out_ref[...] = jnp.max(n * r, axis=1)


@jax.jit
def kernel(logits, uniform):
    B, D = logits.shape
    K = uniform.shape[1]
    bb = 8
    grid = (B // bb,)
    return pl.pallas_call(
        _body,
        grid=grid,
        in_specs=[
            pl.BlockSpec((bb, D), lambda i: (i, 0)),
            pl.BlockSpec((bb, K, D), lambda i: (i, 0, 0)),
        ],
        out_specs=pl.BlockSpec((bb, D), lambda i: (i, 0)),
        out_shape=jax.ShapeDtypeStruct((B, D), logits.dtype),
        compiler_params=pltpu.CompilerParams(
            dimension_semantics=("arbitrary",),
        ),
    )(logits, uniform)
